# baseline (speedup 1.0000x reference)
import jax
import jax.numpy as jnp
from jax import lax
from jax.experimental import pallas as pl
from jax.experimental.pallas import tpu as pltpu
from jax.experimental.pallas import tpu_sc as plsc

NUM_USERS = 1000000
BATCH = 16384
EMBED_DIM = 64
H1 = 128
H2 = 64
BN_EPS = 1e-5

_INFO = plsc.get_sparse_core_info()
_NC = _INFO.num_cores
_NS = _INFO.num_subcores
_NW = _NC * _NS
_ROWS_PER_W = BATCH // _NW
_CHUNK = 128
_NCHUNK = _ROWS_PER_W // _CHUNK
_LANES = 16


def _gather_body(idx_hbm, embf_hbm, xt_hbm, idx_v, off_v, xt_v, sem):
    wid = lax.axis_index("s") * _NC + lax.axis_index("c")
    base = wid * _ROWS_PER_W
    pltpu.sync_copy(idx_hbm.at[pl.ds(base, _ROWS_PER_W)], idx_v)

    def per_feature(f, carry):
        def to_off(j, c):
            v = idx_v[pl.ds(j * _LANES, _LANES)]
            off_v[pl.ds(j * _LANES, _LANES)] = v + f * NUM_USERS
            return c

        lax.fori_loop(0, _ROWS_PER_W // _LANES, to_off, 0)
        for k in range(_NCHUNK):
            pltpu.async_copy(
                embf_hbm.at[off_v.at[pl.ds(k * _CHUNK, _CHUNK)]],
                xt_v.at[f, pl.ds(k * _CHUNK, _CHUNK)],
                sem,
            )
        for k in range(_NCHUNK):
            pltpu.make_async_copy(
                embf_hbm.at[off_v.at[pl.ds(k * _CHUNK, _CHUNK)]],
                xt_v.at[f, pl.ds(k * _CHUNK, _CHUNK)],
                sem,
            ).wait()
        return carry

    lax.fori_loop(0, EMBED_DIM, per_feature, 0)
    pltpu.sync_copy(xt_v, xt_hbm.at[:, pl.ds(base, _ROWS_PER_W)])


_gather = pl.kernel(
    _gather_body,
    out_type=jax.ShapeDtypeStruct((EMBED_DIM, BATCH), jnp.float32),
    mesh=plsc.VectorSubcoreMesh(core_axis_name="c", subcore_axis_name="s"),
    scratch_types=[
        pltpu.VMEM((_ROWS_PER_W,), jnp.int32),
        pltpu.VMEM((_ROWS_PER_W,), jnp.int32),
        pltpu.VMEM((EMBED_DIM, _ROWS_PER_W), jnp.float32),
        pltpu.SemaphoreType.DMA,
    ],
    compiler_params=pltpu.CompilerParams(use_tc_tiling_on_sc=False),
)


_BLKN = 1024
_INV = 1.0 / (1.0 + BN_EPS) ** 0.5


def _mlp_body(xt_ref, w1t_ref, b1_ref, g1_ref, be1_ref, w2t_ref, b2_ref,
              g2_ref, be2_ref, ot_ref):
    xt = xt_ref[...]
    ht = jnp.dot(w1t_ref[...], xt, preferred_element_type=jnp.float32)
    ht = ht + b1_ref[...]
    ht = jnp.maximum(ht, 0.0)
    ht = ht * (_INV * g1_ref[...]) + be1_ref[...]
    ot = jnp.dot(w2t_ref[...], ht, preferred_element_type=jnp.float32)
    ot = ot + b2_ref[...]
    ot = jnp.maximum(ot, 0.0)
    ot_ref[...] = ot * (_INV * g2_ref[...]) + be2_ref[...]


def _full(shape):
    return pl.BlockSpec(shape, lambda i: (0,) * len(shape))


_mlp = pl.pallas_call(
    _mlp_body,
    grid=(BATCH // _BLKN,),
    in_specs=[
        pl.BlockSpec((EMBED_DIM, _BLKN), lambda i: (0, i)),
        _full((H1, EMBED_DIM)),
        _full((H1, 1)),
        _full((H1, 1)),
        _full((H1, 1)),
        _full((H2, H1)),
        _full((H2, 1)),
        _full((H2, 1)),
        _full((H2, 1)),
    ],
    out_specs=pl.BlockSpec((H2, _BLKN), lambda i: (0, i)),
    out_shape=jax.ShapeDtypeStruct((H2, BATCH), jnp.float32),
)


@jax.jit
def kernel(user_ids, emb, W1, b1, g1, be1, W2, b2, g2, be2):
    idx = user_ids.astype(jnp.int32)
    embf = emb.T.reshape(NUM_USERS * EMBED_DIM)
    xt = _gather(idx, embf)
    ot = _mlp(
        xt,
        W1.T,
        b1.reshape(H1, 1),
        g1.reshape(H1, 1),
        be1.reshape(H1, 1),
        W2.T,
        b2.reshape(H2, 1),
        g2.reshape(H2, 1),
        be2.reshape(H2, 1),
    )
    return ot.T


# speedup vs baseline: 8.0499x; 8.0499x over previous
import jax
import jax.numpy as jnp
from jax import lax
from jax.experimental import pallas as pl
from jax.experimental.pallas import tpu as pltpu
from jax.experimental.pallas import tpu_sc as plsc

NUM_USERS = 1000000
BATCH = 16384
EMBED_DIM = 64
H1 = 128
H2 = 64
BN_EPS = 1e-5

_INFO = plsc.get_sparse_core_info()
_NC = _INFO.num_cores
_NS = _INFO.num_subcores
_NW = _NC * _NS
_ROWS_PER_W = BATCH // _NW
_CHUNK = 128
_NCHUNK = _ROWS_PER_W // _CHUNK


def _gather_body(idx_hbm, emb_hbm, x_hbm, idx_v, rows_v, sem):
    wid = lax.axis_index("s") * _NC + lax.axis_index("c")
    base = wid * _ROWS_PER_W
    pltpu.sync_copy(idx_hbm.at[pl.ds(base, _ROWS_PER_W)], idx_v)
    for k in range(_NCHUNK):
        pltpu.async_copy(
            emb_hbm.at[idx_v.at[pl.ds(k * _CHUNK, _CHUNK)]],
            rows_v.at[pl.ds(k * _CHUNK, _CHUNK)],
            sem,
        )
    for k in range(_NCHUNK):
        pltpu.make_async_copy(
            emb_hbm.at[idx_v.at[pl.ds(k * _CHUNK, _CHUNK)]],
            rows_v.at[pl.ds(k * _CHUNK, _CHUNK)],
            sem,
        ).wait()
    pltpu.sync_copy(rows_v, x_hbm.at[pl.ds(base, _ROWS_PER_W)])


_gather = pl.kernel(
    _gather_body,
    out_type=jax.ShapeDtypeStruct((BATCH, EMBED_DIM), jnp.float32),
    mesh=plsc.VectorSubcoreMesh(core_axis_name="c", subcore_axis_name="s"),
    scratch_types=[
        pltpu.VMEM((_ROWS_PER_W,), jnp.int32),
        pltpu.VMEM((_ROWS_PER_W, EMBED_DIM), jnp.float32),
        pltpu.SemaphoreType.DMA,
    ],
    compiler_params=pltpu.CompilerParams(use_tc_tiling_on_sc=False),
)


_BLKB = 4096
_INV = 1.0 / (1.0 + BN_EPS) ** 0.5


def _mlp_body(x_ref, w1_ref, b1_ref, g1_ref, be1_ref, w2_ref, b2_ref,
              g2_ref, be2_ref, o_ref):
    x = x_ref[...]
    h = jnp.dot(x, w1_ref[...], preferred_element_type=jnp.float32)
    h = h + b1_ref[...]
    h = jnp.maximum(h, 0.0)
    h = h * (_INV * g1_ref[...]) + be1_ref[...]
    o = jnp.dot(h, w2_ref[...], preferred_element_type=jnp.float32)
    o = o + b2_ref[...]
    o = jnp.maximum(o, 0.0)
    o_ref[...] = o * (_INV * g2_ref[...]) + be2_ref[...]


def _full(shape):
    return pl.BlockSpec(shape, lambda i: (0,) * len(shape))


_mlp = pl.pallas_call(
    _mlp_body,
    grid=(BATCH // _BLKB,),
    in_specs=[
        pl.BlockSpec((_BLKB, EMBED_DIM), lambda i: (i, 0)),
        _full((EMBED_DIM, H1)),
        _full((1, H1)),
        _full((1, H1)),
        _full((1, H1)),
        _full((H1, H2)),
        _full((1, H2)),
        _full((1, H2)),
        _full((1, H2)),
    ],
    out_specs=pl.BlockSpec((_BLKB, H2), lambda i: (i, 0)),
    out_shape=jax.ShapeDtypeStruct((BATCH, H2), jnp.float32),
)


@jax.jit
def kernel(user_ids, emb, W1, b1, g1, be1, W2, b2, g2, be2):
    idx = user_ids.astype(jnp.int32)
    x = _gather(idx, emb)
    return _mlp(
        x,
        W1,
        b1.reshape(1, H1),
        g1.reshape(1, H1),
        be1.reshape(1, H1),
        W2,
        b2.reshape(1, H2),
        g2.reshape(1, H2),
        be2.reshape(1, H2),
    )
